# baseline (speedup 1.0000x reference)
import jax
import jax.numpy as jnp
from jax.experimental import pallas as pl
from jax.experimental.pallas import tpu as pltpu

_TM = 1024
_KS = 2


def _mean_body(w_ref, b_ref, wb_ref):
    D = w_ref.shape[1]
    wb_ref[:D, :] = (w_ref[0] + w_ref[1] + w_ref[2] + w_ref[3]) * 0.25
    bm = (b_ref[0] + b_ref[1] + b_ref[2] + b_ref[3]) * 0.25
    wb_ref[D:, :] = jnp.broadcast_to(bm, wb_ref[D:, :].shape)


def _gemm_body(x_ref, wb_ref, o_ref):
    j = pl.program_id(1)
    D = _KS * x_ref.shape[1]
    kd = x_ref.shape[1]
    part = jnp.dot(
        x_ref[...],
        wb_ref[pl.ds(j * kd, kd), :],
        preferred_element_type=jnp.float32,
    )

    @pl.when(j == 0)
    def _first():
        o_ref[...] = part

    @pl.when(j == _KS - 1)
    def _last():
        o_ref[...] += part + wb_ref[D, :]

    if _KS > 2:
        @pl.when(jnp.logical_and(j > 0, j < _KS - 1))
        def _mid():
            o_ref[...] += part


def kernel(x, W, b):
    T, D = x.shape
    R, _, E = W.shape
    wb = pl.pallas_call(
        _mean_body,
        in_specs=[
            pl.BlockSpec((R, D, E), lambda: (0, 0, 0)),
            pl.BlockSpec((R, E), lambda: (0, 0)),
        ],
        out_specs=pl.BlockSpec((D + 8, E), lambda: (0, 0)),
        out_shape=jax.ShapeDtypeStruct((D + 8, E), jnp.float32),
    )(W, b)
    return pl.pallas_call(
        _gemm_body,
        grid=(T // _TM, _KS),
        in_specs=[
            pl.BlockSpec((_TM, D // _KS), lambda i, j: (i, j)),
            pl.BlockSpec((D + 8, E), lambda i, j: (0, 0)),
        ],
        out_specs=pl.BlockSpec((_TM, E), lambda i, j: (i, 0)),
        out_shape=jax.ShapeDtypeStruct((T, E), jnp.float32),
        compiler_params=pltpu.CompilerParams(
            dimension_semantics=("arbitrary", "arbitrary"),
        ),
    )(x, wb)
